# baseline (speedup 1.0000x reference)
import jax
import jax.numpy as jnp
from jax import lax
from jax.experimental import pallas as pl
from jax.experimental.pallas import tpu as pltpu


_ROWS = 32 * 1 * 1024
_COLS = 256
_CH_ROWS = 2048
_NCHUNK = _ROWS // _CH_ROWS
_NBUF = 4


def _loss_body(h_hbm, s_hbm, logsum_ref, count_ref, h_v, s_v, hsems, ssems):
    def copies(ci, b):
        sl = pl.ds(ci * _CH_ROWS, _CH_ROWS)
        return (
            pltpu.make_async_copy(h_hbm.at[sl], h_v.at[b], hsems.at[b]),
            pltpu.make_async_copy(s_hbm.at[sl], s_v.at[b], ssems.at[b]),
        )

    for b in range(_NBUF):
        for c in copies(b, b):
            c.start()

    def ring_group(g, carry):
        acc, cnt = carry
        for b in range(_NBUF):
            ci = g * _NBUF + b
            for c in copies(ci, b):
                c.wait()
            h = h_v[b]
            s = s_v[b]
            acc += jnp.sum(h * jnp.log(s))
            cnt += jnp.sum(h)

            @pl.when(ci + _NBUF < _NCHUNK)
            def _prefetch():
                for c in copies(ci + _NBUF, b):
                    c.start()
        return acc, cnt

    acc, cnt = lax.fori_loop(
        0, _NCHUNK // _NBUF, ring_group, (jnp.float32(0.0), jnp.float32(0.0)))
    logsum_ref[...] = jnp.reshape(acc, (1, 1))
    count_ref[...] = jnp.reshape(cnt, (1, 1))


def kernel(hard_attention, soft_attention):
    h2 = hard_attention.reshape(_ROWS, _COLS)
    s2 = soft_attention.reshape(_ROWS, _COLS)
    logsum, count = pl.pallas_call(
        _loss_body,
        in_specs=[
            pl.BlockSpec(memory_space=pltpu.MemorySpace.HBM),
            pl.BlockSpec(memory_space=pltpu.MemorySpace.HBM),
        ],
        out_specs=[
            pl.BlockSpec(memory_space=pltpu.MemorySpace.VMEM),
            pl.BlockSpec(memory_space=pltpu.MemorySpace.VMEM),
        ],
        out_shape=[
            jax.ShapeDtypeStruct((1, 1), jnp.float32),
            jax.ShapeDtypeStruct((1, 1), jnp.float32),
        ],
        scratch_shapes=[
            pltpu.VMEM((_NBUF, _CH_ROWS, _COLS), jnp.float32),
            pltpu.VMEM((_NBUF, _CH_ROWS, _COLS), jnp.float32),
            pltpu.SemaphoreType.DMA((_NBUF,)),
            pltpu.SemaphoreType.DMA((_NBUF,)),
        ],
    )(h2, s2)
    return -logsum[0, 0] / count[0, 0]


# speedup vs baseline: 1.0352x; 1.0352x over previous
import jax
import jax.numpy as jnp
from jax import lax
from jax.experimental import pallas as pl
from jax.experimental.pallas import tpu as pltpu


_ROWS = 32 * 1 * 1024
_COLS = 256
_CH_ROWS = 1024
_NCHUNK = _ROWS // _CH_ROWS
_NBUF = 8


def _loss_body(h_hbm, s_hbm, logsum_ref, count_ref, h_v, s_v, hsems, ssems):
    def copies(ci, b):
        sl = pl.ds(ci * _CH_ROWS, _CH_ROWS)
        return (
            pltpu.make_async_copy(h_hbm.at[sl], h_v.at[b], hsems.at[b]),
            pltpu.make_async_copy(s_hbm.at[sl], s_v.at[b], ssems.at[b]),
        )

    for b in range(_NBUF):
        for c in copies(b, b):
            c.start()

    def ring_group(g, carry):
        acc, cnt = carry
        for b in range(_NBUF):
            ci = g * _NBUF + b
            for c in copies(ci, b):
                c.wait()
            h = h_v[b]
            s = s_v[b]
            acc += jnp.sum(h * jnp.log(s))
            cnt += jnp.sum(h)

            @pl.when(ci + _NBUF < _NCHUNK)
            def _prefetch():
                for c in copies(ci + _NBUF, b):
                    c.start()
        return acc, cnt

    acc, cnt = lax.fori_loop(
        0, _NCHUNK // _NBUF, ring_group, (jnp.float32(0.0), jnp.float32(0.0)))
    logsum_ref[...] = jnp.reshape(acc, (1, 1))
    count_ref[...] = jnp.reshape(cnt, (1, 1))


def kernel(hard_attention, soft_attention):
    h2 = hard_attention.reshape(_ROWS, _COLS)
    s2 = soft_attention.reshape(_ROWS, _COLS)
    logsum, count = pl.pallas_call(
        _loss_body,
        in_specs=[
            pl.BlockSpec(memory_space=pltpu.MemorySpace.HBM),
            pl.BlockSpec(memory_space=pltpu.MemorySpace.HBM),
        ],
        out_specs=[
            pl.BlockSpec(memory_space=pltpu.MemorySpace.VMEM),
            pl.BlockSpec(memory_space=pltpu.MemorySpace.VMEM),
        ],
        out_shape=[
            jax.ShapeDtypeStruct((1, 1), jnp.float32),
            jax.ShapeDtypeStruct((1, 1), jnp.float32),
        ],
        scratch_shapes=[
            pltpu.VMEM((_NBUF, _CH_ROWS, _COLS), jnp.float32),
            pltpu.VMEM((_NBUF, _CH_ROWS, _COLS), jnp.float32),
            pltpu.SemaphoreType.DMA((_NBUF,)),
            pltpu.SemaphoreType.DMA((_NBUF,)),
        ],
    )(h2, s2)
    return -logsum[0, 0] / count[0, 0]
